# Initial kernel scaffold; baseline (speedup 1.0000x reference)
import jax
import jax.numpy as jnp
from jax.experimental import pallas as pl


def kernel(x, lut):
    raise NotImplementedError("write your pallas kernel here")


# speedup vs baseline: 4.5225x; 4.5225x over previous
import functools
import math

import jax
import jax.numpy as jnp
from jax import lax
from jax.experimental import pallas as pl
from jax.experimental.pallas import tpu as pltpu
from jax.experimental.pallas import tpu_sc as plsc

VOCAB = 4634
EMBED_DIM = 64
SCALE = math.sqrt(EMBED_DIM)

_info = plsc.get_sparse_core_info()
_NC = _info.num_cores
_NS = _info.num_subcores
_NW = _NC * _NS


def _scale_body(lut_ref, out_ref):
    out_ref[...] = lut_ref[...] * SCALE


def _make_gather(total, chunk):
    assert total % (_NW * chunk) == 0 and chunk % 8 == 0
    per_worker = total // _NW
    n_chunks = per_worker // chunk
    mesh = plsc.VectorSubcoreMesh(core_axis_name="c", subcore_axis_name="s")

    @functools.partial(
        pl.kernel,
        mesh=mesh,
        out_type=jax.ShapeDtypeStruct((total, EMBED_DIM), jnp.float32),
        scratch_types=[
            pltpu.VMEM((chunk,), jnp.int32),
            pltpu.VMEM((chunk, EMBED_DIM), jnp.float32),
            pltpu.SemaphoreType.DMA,
        ],
        compiler_params=pltpu.CompilerParams(use_tc_tiling_on_sc=False),
    )
    def gather_kernel(table_hbm, idx_hbm, out_hbm, idx_v, rows_v, sem):
        wid = lax.axis_index("s") * _NC + lax.axis_index("c")
        base = wid * per_worker

        def body(i, carry):
            off = base + i * chunk
            pltpu.sync_copy(idx_hbm.at[pl.ds(off, chunk)], idx_v)
            pltpu.async_copy(table_hbm.at[idx_v], rows_v, sem).wait()
            pltpu.sync_copy(rows_v, out_hbm.at[pl.ds(off, chunk)])
            return carry

        lax.fori_loop(0, n_chunks, body, 0)

    return gather_kernel


_gather = _make_gather(4096 * 200, 512)


def kernel(x, lut):
    scaled = pl.pallas_call(
        _scale_body,
        out_shape=jax.ShapeDtypeStruct((VOCAB, EMBED_DIM), jnp.float32),
    )(lut)
    x_flat = x.reshape(-1).astype(jnp.int32)
    out_flat = _gather(scaled, x_flat)
    return out_flat.reshape(x.shape + (EMBED_DIM,))


# speedup vs baseline: 4.8017x; 1.0617x over previous
import functools
import math

import jax
import jax.numpy as jnp
from jax import lax
from jax.experimental import pallas as pl
from jax.experimental.pallas import tpu as pltpu
from jax.experimental.pallas import tpu_sc as plsc

VOCAB = 4634
EMBED_DIM = 64
SCALE = math.sqrt(EMBED_DIM)

_info = plsc.get_sparse_core_info()
_NC = _info.num_cores
_NS = _info.num_subcores
_NW = _NC * _NS


def _scale_body(lut_ref, out_ref):
    out_ref[...] = lut_ref[...] * SCALE


def _make_gather(total, chunk):
    assert total % (_NW * chunk) == 0 and chunk % 8 == 0
    per_worker = total // _NW
    n_chunks = per_worker // chunk
    assert n_chunks % 2 == 0 and n_chunks >= 4
    mesh = plsc.VectorSubcoreMesh(core_axis_name="c", subcore_axis_name="s")

    @functools.partial(
        pl.kernel,
        mesh=mesh,
        out_type=jax.ShapeDtypeStruct((total, EMBED_DIM), jnp.float32),
        scratch_types=[
            pltpu.VMEM((2, chunk), jnp.int32),
            pltpu.VMEM((2, chunk, EMBED_DIM), jnp.float32),
            [pltpu.SemaphoreType.DMA] * 2,
            [pltpu.SemaphoreType.DMA] * 2,
            pltpu.SemaphoreType.DMA,
        ],
        compiler_params=pltpu.CompilerParams(use_tc_tiling_on_sc=False),
    )
    def gather_kernel(table_hbm, idx_hbm, out_hbm, idx_v, rows_v, isem, osem, gsem):
        wid = lax.axis_index("s") * _NC + lax.axis_index("c")
        base = wid * per_worker

        for b in range(2):
            pltpu.async_copy(
                idx_hbm.at[pl.ds(base + b * chunk, chunk)], idx_v.at[b], isem[b]
            )

        def body(j, carry):
            for b in range(2):
                off = base + (2 * j + b) * chunk
                pltpu.make_async_copy(
                    idx_hbm.at[pl.ds(off, chunk)], idx_v.at[b], isem[b]
                ).wait()
                @pl.when(j >= 1)
                def _():
                    pltpu.make_async_copy(
                        rows_v.at[b],
                        out_hbm.at[pl.ds(off - 2 * chunk, chunk)],
                        osem[b],
                    ).wait()

                pltpu.async_copy(table_hbm.at[idx_v.at[b]], rows_v.at[b], gsem).wait()
                @pl.when(j < n_chunks // 2 - 1)
                def _():
                    pltpu.async_copy(
                        idx_hbm.at[pl.ds(off + 2 * chunk, chunk)],
                        idx_v.at[b],
                        isem[b],
                    )

                pltpu.async_copy(
                    rows_v.at[b], out_hbm.at[pl.ds(off, chunk)], osem[b]
                )
            return carry

        lax.fori_loop(0, n_chunks // 2, body, 0)

        for b in range(2):
            off = base + (n_chunks - 2 + b) * chunk
            pltpu.make_async_copy(
                rows_v.at[b], out_hbm.at[pl.ds(off, chunk)], osem[b]
            ).wait()

    return gather_kernel


_gather = _make_gather(4096 * 200, 800)


def kernel(x, lut):
    scaled = pl.pallas_call(
        _scale_body,
        out_shape=jax.ShapeDtypeStruct((VOCAB, EMBED_DIM), jnp.float32),
    )(lut)
    x_flat = x.reshape(-1).astype(jnp.int32)
    out_flat = _gather(scaled, x_flat)
    return out_flat.reshape(x.shape + (EMBED_DIM,))
